# baseline (speedup 1.0000x reference)
import functools

import jax
import jax.numpy as jnp
from jax import lax
from jax.experimental import pallas as pl
from jax.experimental.pallas import tpu as pltpu
from jax.experimental.pallas import tpu_sc as plsc

N = 10000
D = 128
NP = 10240
NC = 2
NS = 16
NW = NC * NS
B = 128
STRIPE = NP // NS
ROWS_TC = 1280
Q0 = 116
Q1 = 44


def _sc_degree(col3, ew3, z1, c_ch):
  mesh = plsc.VectorSubcoreMesh(core_axis_name="c", subcore_axis_name="s")

  @functools.partial(
      pl.kernel,
      out_type=jax.ShapeDtypeStruct((NC, NP), jnp.float32),
      mesh=mesh,
      scratch_types=[
          pltpu.VMEM((c_ch, B), jnp.int32),
          pltpu.VMEM((c_ch, B), jnp.float32),
          pltpu.VMEM_SHARED((NP,), jnp.float32),
      ],
      compiler_params=pltpu.CompilerParams(needs_layout_passes=False),
  )
  def deg_kernel(col_h, ew_h, z_h, out_h, col_v, ew_v, acc):
    cid = lax.axis_index("c")
    sid = lax.axis_index("s")
    wid = cid * NS + sid
    pltpu.sync_copy(col_h.at[wid], col_v)
    pltpu.sync_copy(ew_h.at[wid], ew_v)
    pltpu.sync_copy(z_h, acc.at[pl.ds(sid * STRIPE, STRIPE)])
    plsc.subcore_barrier()

    def body(ci, carry):
      pltpu.sync_copy(ew_v.at[ci], acc.at[col_v.at[ci]], add=True)
      return carry

    lax.fori_loop(0, c_ch, body, 0)
    plsc.subcore_barrier()
    pltpu.sync_copy(acc.at[pl.ds(sid * STRIPE, STRIPE)],
                    out_h.at[cid, pl.ds(sid * STRIPE, STRIPE)])

  return deg_kernel(col3, ew3, z1)


def _sc_propagate(lin, pk, dis_flat, z2, q0, q1):
  mesh = plsc.VectorSubcoreMesh(core_axis_name="c", subcore_axis_name="s")

  @functools.partial(
      pl.kernel,
      out_type=jax.ShapeDtypeStruct((NC, NP, D), jnp.float32),
      mesh=mesh,
      scratch_types=[
          pltpu.VMEM((12, B), jnp.int32),
          pltpu.VMEM((NP,), jnp.float32),
          pltpu.VMEM((B,), jnp.float32),
          pltpu.VMEM((2, B, D), jnp.float32),
          pltpu.VMEM_SHARED((NP, D), jnp.float32),
          pltpu.SemaphoreType.DMA,
          pltpu.SemaphoreType.DMA,
          pltpu.SemaphoreType.DMA,
      ],
      compiler_params=pltpu.CompilerParams(needs_layout_passes=False),
  )
  def spmm_kernel(lin_h, pk_h, dis_h, z_h, out_h,
                  pk_v, dis_v, nrm_v, rows_v, acc, isem, gsem, ssem):
    cid = lax.axis_index("c")
    sid = lax.axis_index("s")
    wid = cid * NS + sid
    myc = jnp.where(cid == 0, q0, q1)
    pltpu.sync_copy(dis_h, dis_v)
    pltpu.sync_copy(z_h, acc.at[pl.ds(sid * STRIPE, STRIPE)])
    plsc.subcore_barrier()

    def wait_bytes(dst_ref, sem):
      pltpu.make_async_copy(z_h.at[pl.ds(0, B)], dst_ref, sem).wait()

    def wait_idx(slot):
      pltpu.make_async_copy(pk_h.at[wid, 0],
                            pk_v.at[pl.ds(3 * slot, 3)], isem).wait()

    pltpu.sync_copy(pk_h.at[wid, 0], pk_v.at[pl.ds(0, 3)])
    pltpu.async_copy(lin_h.at[pk_v.at[0]], rows_v.at[0], gsem)
    pltpu.async_copy(pk_h.at[wid, 1], pk_v.at[pl.ds(3, 3)], isem)

    def outer(j, carry):
      for b in range(4):
        ci = 4 * j + b
        rb = b % 2
        nrb = 1 - rb
        nslot = (b + 1) % 4
        wait_bytes(rows_v.at[rb], gsem)
        @pl.when(ci >= 1)
        def _():
          wait_bytes(rows_v.at[nrb], ssem)
        @pl.when(ci + 1 < myc)
        def _():
          wait_idx(nslot)
          pltpu.async_copy(lin_h.at[pk_v.at[3 * nslot]], rows_v.at[nrb], gsem)
        @pl.when(ci + 2 < myc)
        def _():
          pltpu.async_copy(pk_h.at[wid, ci + 2],
                           pk_v.at[pl.ds(3 * ((b + 2) % 4), 3)], isem)
        for g in range(B // 16):
          r16 = pk_v[3 * b, pl.ds(g * 16, 16)]
          c16 = pk_v[3 * b + 1, pl.ds(g * 16, 16)]
          e16 = plsc.bitcast(pk_v[3 * b + 2, pl.ds(g * 16, 16)], jnp.float32)
          n16 = (plsc.load_gather(dis_v, [r16]) * e16
                 * plsc.load_gather(dis_v, [c16]))
          nrm_v[pl.ds(g * 16, 16)] = n16

        def edge(t, c2):
          for k in range(4):
            i = 4 * t + k
            nb = plsc.load_gather(nrm_v, [jnp.zeros((16,), jnp.int32) + i])
            for f in range(D // 16):
              rows_v[rb, i, pl.ds(f * 16, 16)] = (
                  rows_v[rb, i, pl.ds(f * 16, 16)] * nb)
          return c2

        lax.fori_loop(0, B // 4, edge, 0)
        pltpu.async_copy(rows_v.at[rb], acc.at[pk_v.at[3 * b + 1]], ssem,
                         add=True)
      return carry

    lax.fori_loop(0, myc // 4, outer, 0)
    wait_bytes(rows_v.at[0], ssem)
    plsc.subcore_barrier()
    pltpu.sync_copy(acc.at[pl.ds(sid * STRIPE, STRIPE)],
                    out_h.at[cid, pl.ds(sid * STRIPE, STRIPE)])

  return spmm_kernel(lin, pk, dis_flat, z2)


def _tc_lin_dis(x_p, w1, degp):
  grid = (NP // ROWS_TC,)

  def body(x_ref, w_ref, deg_ref, lin_ref, dis_ref):
    i = pl.program_id(0)
    lin_ref[...] = lax.dot_general(
        x_ref[...].astype(jnp.bfloat16), w_ref[...].astype(jnp.bfloat16),
        (((1,), (1,)), ((), ())),
        preferred_element_type=jnp.float32)

    @pl.when(i == 0)
    def _():
      dg = deg_ref[0] + deg_ref[1] + 1.0
      dis_ref[...] = jnp.where(dg > 0, 1.0 / jnp.sqrt(dg), 0.0)

  return pl.pallas_call(
      body,
      grid=grid,
      in_specs=[
          pl.BlockSpec((ROWS_TC, D), lambda i: (i, 0)),
          pl.BlockSpec((D, D), lambda i: (0, 0)),
          pl.BlockSpec((NC, NP // 128, 128), lambda i: (0, 0, 0)),
      ],
      out_specs=[
          pl.BlockSpec((ROWS_TC, D), lambda i: (i, 0)),
          pl.BlockSpec((NP // 128, 128), lambda i: (0, 0)),
      ],
      out_shape=[
          jax.ShapeDtypeStruct((NP, D), jnp.float32),
          jax.ShapeDtypeStruct((NP // 128, 128), jnp.float32),
      ],
  )(x_p, w1, degp)


def _tc_combine_matmul(p0, p1, lin, dis_col, b_2d, w):
  grid = (NP // ROWS_TC,)

  def body(p0_ref, p1_ref, lin_ref, sn_ref, b_ref, w_ref, out_ref):
    sn = sn_ref[...]
    h = p0_ref[...] + p1_ref[...] + lin_ref[...] * (sn * sn) + b_ref[...]
    h = jnp.maximum(h, 0.0)
    out_ref[...] = lax.dot_general(
        h.astype(jnp.bfloat16), w_ref[...].astype(jnp.bfloat16),
        (((1,), (1,)), ((), ())),
        preferred_element_type=jnp.float32)

  return pl.pallas_call(
      body,
      grid=grid,
      in_specs=[
          pl.BlockSpec((ROWS_TC, D), lambda i: (i, 0)),
          pl.BlockSpec((ROWS_TC, D), lambda i: (i, 0)),
          pl.BlockSpec((ROWS_TC, D), lambda i: (i, 0)),
          pl.BlockSpec((ROWS_TC, 1), lambda i: (i, 0)),
          pl.BlockSpec((1, D), lambda i: (0, 0)),
          pl.BlockSpec((D, D), lambda i: (0, 0)),
      ],
      out_specs=pl.BlockSpec((ROWS_TC, D), lambda i: (i, 0)),
      out_shape=jax.ShapeDtypeStruct((NP, D), jnp.float32),
  )(p0, p1, lin, dis_col, b_2d, w)


def _tc_final(p0, p1, lin2, dis_col, b_2d, cw2, cb_2d, fw, fb_2d):
  def body(p0_ref, p1_ref, lin_ref, sn_ref, b_ref, cw_ref, cb_ref, fw_ref,
           fb_ref, out_ref):
    sn = sn_ref[...]
    h = p0_ref[...] + p1_ref[...] + lin_ref[...] * (sn * sn) + b_ref[...]
    h = jnp.maximum(h, 0.0)
    ridx = lax.broadcasted_iota(jnp.int32, (NP, D), 0)
    h = jnp.where(ridx < N, h, 0.0)
    h_prev = jnp.where(ridx == 0, 0.0, pltpu.roll(h, 1, 0))
    h_next = jnp.where(ridx == NP - 1, 0.0, pltpu.roll(h, NP - 1, 0))
    hs = (h_prev, h, h_next)
    t = cb_ref[...].astype(jnp.float32)
    for k in range(3):
      t = t + lax.dot_general(
          hs[k].astype(jnp.bfloat16),
          cw_ref[k * D:(k + 1) * D, :].astype(jnp.bfloat16),
          (((1,), (1,)), ((), ())), preferred_element_type=jnp.float32)
    out_ref[...] = lax.dot_general(
        t.astype(jnp.bfloat16), fw_ref[...].astype(jnp.bfloat16),
        (((1,), (1,)), ((), ())),
        preferred_element_type=jnp.float32) + fb_ref[0, 0]

  return pl.pallas_call(
      body,
      out_shape=jax.ShapeDtypeStruct((NP, 8), jnp.float32),
  )(p0, p1, lin2, dis_col, b_2d, cw2, cb_2d, fw, fb_2d)


def kernel(x, edge_index, edge_weight, W1, b1, W2, b2, conv_w, conv_b,
           fc_w, fc_b):
  e_cnt = edge_weight.shape[0]
  c_ch = 4 * (-(-e_cnt // (NW * B * 4)))
  ept = c_ch * B
  pad = ept * NW - e_cnt

  row = edge_index[0].astype(jnp.int32)
  col = edge_index[1].astype(jnp.int32)
  ew = edge_weight.astype(jnp.float32)
  row3 = jnp.pad(row, (0, pad)).reshape(NW, c_ch, B)
  col3 = jnp.pad(col, (0, pad)).reshape(NW, c_ch, B)
  ew3 = jnp.pad(ew, (0, pad)).reshape(NW, c_ch, B)

  cap = NS * (Q0 + Q1) * B
  cmax = max(Q0, Q1)

  def _split(a):
    ap = jnp.pad(a, (0, cap - e_cnt))
    a0 = ap[:NS * Q0 * B].reshape(NS, Q0, B)
    a1 = ap[NS * Q0 * B:].reshape(NS, Q1, B)
    a0 = jnp.pad(a0, ((0, 0), (0, cmax - Q0), (0, 0)))
    a1 = jnp.pad(a1, ((0, 0), (0, cmax - Q1), (0, 0)))
    return jnp.concatenate([a0, a1], axis=0)

  pk = jnp.stack(
      [_split(row), _split(col),
       _split(lax.bitcast_convert_type(ew, jnp.int32))], axis=2)
  x_p = jnp.pad(x, ((0, NP - N), (0, 0)))
  z2 = jnp.zeros((STRIPE, D), jnp.float32)
  z1 = jnp.zeros((STRIPE,), jnp.float32)

  _DBG = 0

  def _jnp_prop(lin):
    nrm = dis_flat[row] * ew * dis_flat[col]
    p = jnp.zeros((NP, D), jnp.float32).at[col].add(lin[row] * nrm[:, None])
    return jnp.stack([p, jnp.zeros_like(p)])

  degp = _sc_degree(col3, ew3, z1, c_ch)
  if _DBG in (2, 3):
    dg = jnp.zeros((NP,), jnp.float32).at[col].add(ew)
    degp = jnp.stack([dg, jnp.zeros_like(dg)])
  lin1, dis2d = _tc_lin_dis(x_p, W1, degp.reshape(NC, NP // 128, 128))
  dis_flat = dis2d.reshape(NP)
  dis_col = dis2d.reshape(NP, 1)

  if _DBG in (1, 3):
    part1 = _jnp_prop(lin1)
  else:
    part1 = _sc_propagate(lin1, pk, dis_flat, z2, Q0, Q1)
  lin2 = _tc_combine_matmul(part1[0], part1[1], lin1, dis_col,
                            b1.reshape(1, D), W2)
  if _DBG in (1, 3):
    part2 = _jnp_prop(lin2)
  else:
    part2 = _sc_propagate(lin2, pk, dis_flat, z2, Q0, Q1)
  cw2 = jnp.moveaxis(conv_w, 2, 0).reshape(3 * D, D)
  fw8 = jnp.pad(fc_w, ((0, 7), (0, 0)))
  res = _tc_final(part2[0], part2[1], lin2, dis_col, b2.reshape(1, D),
                  cw2, conv_b.reshape(1, D), fw8, fc_b.reshape(1, 1))
  return res[:N, 0]


# speedup vs baseline: 1.0525x; 1.0525x over previous
import functools

import jax
import jax.numpy as jnp
from jax import lax
from jax.experimental import pallas as pl
from jax.experimental.pallas import tpu as pltpu
from jax.experimental.pallas import tpu_sc as plsc

N = 10000
D = 128
NP = 10240
NC = 2
NS = 16
NW = NC * NS
B = 128
STRIPE = NP // NS
ROWS_TC = 1280
Q0 = 44
Q1 = 116


def _sc_degree(col3, ew3, z1, c_ch):
  mesh = plsc.VectorSubcoreMesh(core_axis_name="c", subcore_axis_name="s")

  @functools.partial(
      pl.kernel,
      out_type=jax.ShapeDtypeStruct((NC, NP), jnp.float32),
      mesh=mesh,
      scratch_types=[
          pltpu.VMEM((c_ch, B), jnp.int32),
          pltpu.VMEM((c_ch, B), jnp.float32),
          pltpu.VMEM_SHARED((NP,), jnp.float32),
      ],
      compiler_params=pltpu.CompilerParams(needs_layout_passes=False),
  )
  def deg_kernel(col_h, ew_h, z_h, out_h, col_v, ew_v, acc):
    cid = lax.axis_index("c")
    sid = lax.axis_index("s")
    wid = cid * NS + sid
    pltpu.sync_copy(col_h.at[wid], col_v)
    pltpu.sync_copy(ew_h.at[wid], ew_v)
    pltpu.sync_copy(z_h, acc.at[pl.ds(sid * STRIPE, STRIPE)])
    plsc.subcore_barrier()

    def body(ci, carry):
      pltpu.sync_copy(ew_v.at[ci], acc.at[col_v.at[ci]], add=True)
      return carry

    lax.fori_loop(0, c_ch, body, 0)
    plsc.subcore_barrier()
    pltpu.sync_copy(acc.at[pl.ds(sid * STRIPE, STRIPE)],
                    out_h.at[cid, pl.ds(sid * STRIPE, STRIPE)])

  return deg_kernel(col3, ew3, z1)


def _sc_propagate(lin, pk, dis_flat, z2, q0, q1):
  mesh = plsc.VectorSubcoreMesh(core_axis_name="c", subcore_axis_name="s")

  @functools.partial(
      pl.kernel,
      out_type=jax.ShapeDtypeStruct((NC, NP, D), jnp.float32),
      mesh=mesh,
      scratch_types=[
          pltpu.VMEM((12, B), jnp.int32),
          pltpu.VMEM((NP,), jnp.float32),
          pltpu.VMEM((B,), jnp.float32),
          pltpu.VMEM((2, B, D), jnp.float32),
          pltpu.VMEM_SHARED((NP, D), jnp.float32),
          pltpu.SemaphoreType.DMA,
          pltpu.SemaphoreType.DMA,
          pltpu.SemaphoreType.DMA,
      ],
      compiler_params=pltpu.CompilerParams(needs_layout_passes=False),
  )
  def spmm_kernel(lin_h, pk_h, dis_h, z_h, out_h,
                  pk_v, dis_v, nrm_v, rows_v, acc, isem, gsem, ssem):
    cid = lax.axis_index("c")
    sid = lax.axis_index("s")
    wid = cid * NS + sid
    myc = jnp.where(cid == 0, q0, q1)
    pltpu.sync_copy(dis_h, dis_v)
    pltpu.sync_copy(z_h, acc.at[pl.ds(sid * STRIPE, STRIPE)])
    plsc.subcore_barrier()

    def wait_bytes(dst_ref, sem):
      pltpu.make_async_copy(z_h.at[pl.ds(0, B)], dst_ref, sem).wait()

    def wait_idx(slot):
      pltpu.make_async_copy(pk_h.at[wid, 0],
                            pk_v.at[pl.ds(3 * slot, 3)], isem).wait()

    pltpu.sync_copy(pk_h.at[wid, 0], pk_v.at[pl.ds(0, 3)])
    pltpu.async_copy(lin_h.at[pk_v.at[0]], rows_v.at[0], gsem)
    pltpu.async_copy(pk_h.at[wid, 1], pk_v.at[pl.ds(3, 3)], isem)

    def outer(j, carry):
      for b in range(4):
        ci = 4 * j + b
        rb = b % 2
        nrb = 1 - rb
        nslot = (b + 1) % 4
        wait_bytes(rows_v.at[rb], gsem)
        @pl.when(ci >= 1)
        def _():
          wait_bytes(rows_v.at[nrb], ssem)
        @pl.when(ci + 1 < myc)
        def _():
          wait_idx(nslot)
          pltpu.async_copy(lin_h.at[pk_v.at[3 * nslot]], rows_v.at[nrb], gsem)
        @pl.when(ci + 2 < myc)
        def _():
          pltpu.async_copy(pk_h.at[wid, ci + 2],
                           pk_v.at[pl.ds(3 * ((b + 2) % 4), 3)], isem)
        for g in range(B // 16):
          r16 = pk_v[3 * b, pl.ds(g * 16, 16)]
          c16 = pk_v[3 * b + 1, pl.ds(g * 16, 16)]
          e16 = plsc.bitcast(pk_v[3 * b + 2, pl.ds(g * 16, 16)], jnp.float32)
          n16 = (plsc.load_gather(dis_v, [r16]) * e16
                 * plsc.load_gather(dis_v, [c16]))
          nrm_v[pl.ds(g * 16, 16)] = n16

        def edge(t, c2):
          for k in range(4):
            i = 4 * t + k
            nb = plsc.load_gather(nrm_v, [jnp.zeros((16,), jnp.int32) + i])
            for f in range(D // 16):
              rows_v[rb, i, pl.ds(f * 16, 16)] = (
                  rows_v[rb, i, pl.ds(f * 16, 16)] * nb)
          return c2

        lax.fori_loop(0, B // 4, edge, 0)
        pltpu.async_copy(rows_v.at[rb], acc.at[pk_v.at[3 * b + 1]], ssem,
                         add=True)
      return carry

    lax.fori_loop(0, myc // 4, outer, 0)
    wait_bytes(rows_v.at[0], ssem)
    plsc.subcore_barrier()
    pltpu.sync_copy(acc.at[pl.ds(sid * STRIPE, STRIPE)],
                    out_h.at[cid, pl.ds(sid * STRIPE, STRIPE)])

  return spmm_kernel(lin, pk, dis_flat, z2)


def _tc_lin_dis(x_p, w1, degp):
  grid = (NP // ROWS_TC,)

  def body(x_ref, w_ref, deg_ref, lin_ref, dis_ref):
    i = pl.program_id(0)
    lin_ref[...] = lax.dot_general(
        x_ref[...].astype(jnp.bfloat16), w_ref[...].astype(jnp.bfloat16),
        (((1,), (1,)), ((), ())),
        preferred_element_type=jnp.float32)

    @pl.when(i == 0)
    def _():
      dg = deg_ref[0] + deg_ref[1] + 1.0
      dis_ref[...] = jnp.where(dg > 0, 1.0 / jnp.sqrt(dg), 0.0)

  return pl.pallas_call(
      body,
      grid=grid,
      in_specs=[
          pl.BlockSpec((ROWS_TC, D), lambda i: (i, 0)),
          pl.BlockSpec((D, D), lambda i: (0, 0)),
          pl.BlockSpec((NC, NP // 128, 128), lambda i: (0, 0, 0)),
      ],
      out_specs=[
          pl.BlockSpec((ROWS_TC, D), lambda i: (i, 0)),
          pl.BlockSpec((NP // 128, 128), lambda i: (0, 0)),
      ],
      out_shape=[
          jax.ShapeDtypeStruct((NP, D), jnp.float32),
          jax.ShapeDtypeStruct((NP // 128, 128), jnp.float32),
      ],
  )(x_p, w1, degp)


def _tc_combine_matmul(p0, p1, lin, dis_col, b_2d, w):
  grid = (NP // ROWS_TC,)

  def body(p0_ref, p1_ref, lin_ref, sn_ref, b_ref, w_ref, out_ref):
    sn = sn_ref[...]
    h = p0_ref[...] + p1_ref[...] + lin_ref[...] * (sn * sn) + b_ref[...]
    h = jnp.maximum(h, 0.0)
    out_ref[...] = lax.dot_general(
        h.astype(jnp.bfloat16), w_ref[...].astype(jnp.bfloat16),
        (((1,), (1,)), ((), ())),
        preferred_element_type=jnp.float32)

  return pl.pallas_call(
      body,
      grid=grid,
      in_specs=[
          pl.BlockSpec((ROWS_TC, D), lambda i: (i, 0)),
          pl.BlockSpec((ROWS_TC, D), lambda i: (i, 0)),
          pl.BlockSpec((ROWS_TC, D), lambda i: (i, 0)),
          pl.BlockSpec((ROWS_TC, 1), lambda i: (i, 0)),
          pl.BlockSpec((1, D), lambda i: (0, 0)),
          pl.BlockSpec((D, D), lambda i: (0, 0)),
      ],
      out_specs=pl.BlockSpec((ROWS_TC, D), lambda i: (i, 0)),
      out_shape=jax.ShapeDtypeStruct((NP, D), jnp.float32),
  )(p0, p1, lin, dis_col, b_2d, w)


def _tc_final(p0, p1, lin2, dis_col, b_2d, cw2, cb_2d, fw, fb_2d):
  def body(p0_ref, p1_ref, lin_ref, sn_ref, b_ref, cw_ref, cb_ref, fw_ref,
           fb_ref, out_ref):
    sn = sn_ref[...]
    h = p0_ref[...] + p1_ref[...] + lin_ref[...] * (sn * sn) + b_ref[...]
    h = jnp.maximum(h, 0.0)
    ridx = lax.broadcasted_iota(jnp.int32, (NP, D), 0)
    h = jnp.where(ridx < N, h, 0.0)
    h_prev = jnp.where(ridx == 0, 0.0, pltpu.roll(h, 1, 0))
    h_next = jnp.where(ridx == NP - 1, 0.0, pltpu.roll(h, NP - 1, 0))
    hs = (h_prev, h, h_next)
    t = cb_ref[...].astype(jnp.float32)
    for k in range(3):
      t = t + lax.dot_general(
          hs[k].astype(jnp.bfloat16),
          cw_ref[k * D:(k + 1) * D, :].astype(jnp.bfloat16),
          (((1,), (1,)), ((), ())), preferred_element_type=jnp.float32)
    out_ref[...] = lax.dot_general(
        t.astype(jnp.bfloat16), fw_ref[...].astype(jnp.bfloat16),
        (((1,), (1,)), ((), ())),
        preferred_element_type=jnp.float32) + fb_ref[0, 0]

  return pl.pallas_call(
      body,
      out_shape=jax.ShapeDtypeStruct((NP, 8), jnp.float32),
  )(p0, p1, lin2, dis_col, b_2d, cw2, cb_2d, fw, fb_2d)


def kernel(x, edge_index, edge_weight, W1, b1, W2, b2, conv_w, conv_b,
           fc_w, fc_b):
  e_cnt = edge_weight.shape[0]
  c_ch = 4 * (-(-e_cnt // (NW * B * 4)))
  ept = c_ch * B
  pad = ept * NW - e_cnt

  row = edge_index[0].astype(jnp.int32)
  col = edge_index[1].astype(jnp.int32)
  ew = edge_weight.astype(jnp.float32)
  row3 = jnp.pad(row, (0, pad)).reshape(NW, c_ch, B)
  col3 = jnp.pad(col, (0, pad)).reshape(NW, c_ch, B)
  ew3 = jnp.pad(ew, (0, pad)).reshape(NW, c_ch, B)

  cap = NS * (Q0 + Q1) * B
  cmax = max(Q0, Q1)

  def _split(a):
    ap = jnp.pad(a, (0, cap - e_cnt))
    a0 = ap[:NS * Q0 * B].reshape(NS, Q0, B)
    a1 = ap[NS * Q0 * B:].reshape(NS, Q1, B)
    a0 = jnp.pad(a0, ((0, 0), (0, cmax - Q0), (0, 0)))
    a1 = jnp.pad(a1, ((0, 0), (0, cmax - Q1), (0, 0)))
    return jnp.concatenate([a0, a1], axis=0)

  pk = jnp.stack(
      [_split(row), _split(col),
       _split(lax.bitcast_convert_type(ew, jnp.int32))], axis=2)
  x_p = jnp.pad(x, ((0, NP - N), (0, 0)))
  z2 = jnp.zeros((STRIPE, D), jnp.float32)
  z1 = jnp.zeros((STRIPE,), jnp.float32)

  _DBG = 0

  def _jnp_prop(lin):
    nrm = dis_flat[row] * ew * dis_flat[col]
    p = jnp.zeros((NP, D), jnp.float32).at[col].add(lin[row] * nrm[:, None])
    return jnp.stack([p, jnp.zeros_like(p)])

  degp = _sc_degree(col3, ew3, z1, c_ch)
  if _DBG in (2, 3):
    dg = jnp.zeros((NP,), jnp.float32).at[col].add(ew)
    degp = jnp.stack([dg, jnp.zeros_like(dg)])
  lin1, dis2d = _tc_lin_dis(x_p, W1, degp.reshape(NC, NP // 128, 128))
  dis_flat = dis2d.reshape(NP)
  dis_col = dis2d.reshape(NP, 1)

  if _DBG in (1, 3):
    part1 = _jnp_prop(lin1)
  else:
    part1 = _sc_propagate(lin1, pk, dis_flat, z2, Q0, Q1)
  lin2 = _tc_combine_matmul(part1[0], part1[1], lin1, dis_col,
                            b1.reshape(1, D), W2)
  if _DBG in (1, 3):
    part2 = _jnp_prop(lin2)
  else:
    part2 = _sc_propagate(lin2, pk, dis_flat, z2, Q0, Q1)
  cw2 = jnp.moveaxis(conv_w, 2, 0).reshape(3 * D, D)
  fw8 = jnp.pad(fc_w, ((0, 7), (0, 0)))
  res = _tc_final(part2[0], part2[1], lin2, dis_col, b2.reshape(1, D),
                  cw2, conv_b.reshape(1, D), fw8, fc_b.reshape(1, 1))
  return res[:N, 0]


# speedup vs baseline: 1.0582x; 1.0054x over previous
import functools

import jax
import jax.numpy as jnp
from jax import lax
from jax.experimental import pallas as pl
from jax.experimental.pallas import tpu as pltpu
from jax.experimental.pallas import tpu_sc as plsc

N = 10000
D = 128
NP = 10240
NC = 2
NS = 16
NW = NC * NS
B = 128
STRIPE = NP // NS
ROWS_TC = 1280
Q0 = 44
Q1 = 116


def _sc_degree(col3, ew3, z1, c_ch):
  mesh = plsc.VectorSubcoreMesh(core_axis_name="c", subcore_axis_name="s")

  @functools.partial(
      pl.kernel,
      out_type=jax.ShapeDtypeStruct((NC, NP), jnp.float32),
      mesh=mesh,
      scratch_types=[
          pltpu.VMEM((c_ch, B), jnp.int32),
          pltpu.VMEM((c_ch, B), jnp.float32),
          pltpu.VMEM_SHARED((NP,), jnp.float32),
      ],
      compiler_params=pltpu.CompilerParams(needs_layout_passes=False),
  )
  def deg_kernel(col_h, ew_h, z_h, out_h, col_v, ew_v, acc):
    cid = lax.axis_index("c")
    sid = lax.axis_index("s")
    wid = cid * NS + sid
    pltpu.sync_copy(col_h.at[wid], col_v)
    pltpu.sync_copy(ew_h.at[wid], ew_v)
    pltpu.sync_copy(z_h, acc.at[pl.ds(sid * STRIPE, STRIPE)])
    plsc.subcore_barrier()

    def body(ci, carry):
      pltpu.sync_copy(ew_v.at[ci], acc.at[col_v.at[ci]], add=True)
      return carry

    lax.fori_loop(0, c_ch, body, 0)
    plsc.subcore_barrier()
    pltpu.sync_copy(acc.at[pl.ds(sid * STRIPE, STRIPE)],
                    out_h.at[cid, pl.ds(sid * STRIPE, STRIPE)])

  return deg_kernel(col3, ew3, z1)


def _sc_propagate(lin, pk, dis_flat, z2, q0, q1):
  mesh = plsc.VectorSubcoreMesh(core_axis_name="c", subcore_axis_name="s")

  @functools.partial(
      pl.kernel,
      out_type=jax.ShapeDtypeStruct((NC, NP, D), jnp.float32),
      mesh=mesh,
      scratch_types=[
          pltpu.VMEM((12, B), jnp.int32),
          pltpu.VMEM((NP,), jnp.float32),
          pltpu.VMEM((B,), jnp.float32),
          pltpu.VMEM((2, B, D), jnp.float32),
          pltpu.VMEM_SHARED((NP, D), jnp.float32),
          pltpu.SemaphoreType.DMA,
          pltpu.SemaphoreType.DMA,
          pltpu.SemaphoreType.DMA,
      ],
      compiler_params=pltpu.CompilerParams(needs_layout_passes=False),
  )
  def spmm_kernel(lin_h, pk_h, dis_h, z_h, out_h,
                  pk_v, dis_v, nrm_v, rows_v, acc, isem, gsem, ssem):
    cid = lax.axis_index("c")
    sid = lax.axis_index("s")
    wid = cid * NS + sid
    myc = jnp.where(cid == 0, q0, q1)
    pltpu.sync_copy(dis_h, dis_v)

    def zrow(r, c):
      for f in range(D // 16):
        rows_v[0, r, pl.ds(f * 16, 16)] = jnp.zeros((16,), jnp.float32)
      return c

    lax.fori_loop(0, B, zrow, 0)
    for t in range(STRIPE // B):
      pltpu.sync_copy(rows_v.at[0], acc.at[pl.ds(sid * STRIPE + t * B, B)])
    plsc.subcore_barrier()

    def wait_bytes(dst_ref, sem):
      pltpu.make_async_copy(z_h.at[pl.ds(0, B)], dst_ref, sem).wait()

    def wait_idx(slot):
      pltpu.make_async_copy(pk_h.at[wid, 0],
                            pk_v.at[pl.ds(3 * slot, 3)], isem).wait()

    pltpu.sync_copy(pk_h.at[wid, 0], pk_v.at[pl.ds(0, 3)])
    pltpu.async_copy(lin_h.at[pk_v.at[0]], rows_v.at[0], gsem)
    pltpu.async_copy(pk_h.at[wid, 1], pk_v.at[pl.ds(3, 3)], isem)

    def outer(j, carry):
      for b in range(4):
        ci = 4 * j + b
        rb = b % 2
        nrb = 1 - rb
        nslot = (b + 1) % 4
        wait_bytes(rows_v.at[rb], gsem)
        @pl.when(ci >= 1)
        def _():
          wait_bytes(rows_v.at[nrb], ssem)
        @pl.when(ci + 1 < myc)
        def _():
          wait_idx(nslot)
          pltpu.async_copy(lin_h.at[pk_v.at[3 * nslot]], rows_v.at[nrb], gsem)
        @pl.when(ci + 2 < myc)
        def _():
          pltpu.async_copy(pk_h.at[wid, ci + 2],
                           pk_v.at[pl.ds(3 * ((b + 2) % 4), 3)], isem)
        for g in range(B // 16):
          r16 = pk_v[3 * b, pl.ds(g * 16, 16)]
          c16 = pk_v[3 * b + 1, pl.ds(g * 16, 16)]
          e16 = plsc.bitcast(pk_v[3 * b + 2, pl.ds(g * 16, 16)], jnp.float32)
          n16 = (plsc.load_gather(dis_v, [r16]) * e16
                 * plsc.load_gather(dis_v, [c16]))
          nrm_v[pl.ds(g * 16, 16)] = n16

        def edge(t, c2):
          for k in range(4):
            i = 4 * t + k
            nb = plsc.load_gather(nrm_v, [jnp.zeros((16,), jnp.int32) + i])
            for f in range(D // 16):
              rows_v[rb, i, pl.ds(f * 16, 16)] = (
                  rows_v[rb, i, pl.ds(f * 16, 16)] * nb)
          return c2

        lax.fori_loop(0, B // 4, edge, 0)
        pltpu.async_copy(rows_v.at[rb], acc.at[pk_v.at[3 * b + 1]], ssem,
                         add=True)
      return carry

    lax.fori_loop(0, myc // 4, outer, 0)
    wait_bytes(rows_v.at[0], ssem)
    plsc.subcore_barrier()
    pltpu.sync_copy(acc.at[pl.ds(sid * STRIPE, STRIPE)],
                    out_h.at[cid, pl.ds(sid * STRIPE, STRIPE)])

  return spmm_kernel(lin, pk, dis_flat, z2)


def _tc_lin_dis(x_p, w1, degp):
  grid = (NP // ROWS_TC,)

  def body(x_ref, w_ref, deg_ref, lin_ref, dis_ref):
    i = pl.program_id(0)
    lin_ref[...] = lax.dot_general(
        x_ref[...].astype(jnp.bfloat16), w_ref[...].astype(jnp.bfloat16),
        (((1,), (1,)), ((), ())),
        preferred_element_type=jnp.float32)

    @pl.when(i == 0)
    def _():
      dg = deg_ref[0] + deg_ref[1] + 1.0
      dis_ref[...] = jnp.where(dg > 0, 1.0 / jnp.sqrt(dg), 0.0)

  return pl.pallas_call(
      body,
      grid=grid,
      in_specs=[
          pl.BlockSpec((ROWS_TC, D), lambda i: (i, 0)),
          pl.BlockSpec((D, D), lambda i: (0, 0)),
          pl.BlockSpec((NC, NP // 128, 128), lambda i: (0, 0, 0)),
      ],
      out_specs=[
          pl.BlockSpec((ROWS_TC, D), lambda i: (i, 0)),
          pl.BlockSpec((NP // 128, 128), lambda i: (0, 0)),
      ],
      out_shape=[
          jax.ShapeDtypeStruct((NP, D), jnp.float32),
          jax.ShapeDtypeStruct((NP // 128, 128), jnp.float32),
      ],
  )(x_p, w1, degp)


def _tc_combine_matmul(p0, p1, lin, dis_col, b_2d, w):
  grid = (NP // ROWS_TC,)

  def body(p0_ref, p1_ref, lin_ref, sn_ref, b_ref, w_ref, out_ref):
    sn = sn_ref[...]
    h = p0_ref[...] + p1_ref[...] + lin_ref[...] * (sn * sn) + b_ref[...]
    h = jnp.maximum(h, 0.0)
    out_ref[...] = lax.dot_general(
        h.astype(jnp.bfloat16), w_ref[...].astype(jnp.bfloat16),
        (((1,), (1,)), ((), ())),
        preferred_element_type=jnp.float32)

  return pl.pallas_call(
      body,
      grid=grid,
      in_specs=[
          pl.BlockSpec((ROWS_TC, D), lambda i: (i, 0)),
          pl.BlockSpec((ROWS_TC, D), lambda i: (i, 0)),
          pl.BlockSpec((ROWS_TC, D), lambda i: (i, 0)),
          pl.BlockSpec((ROWS_TC, 1), lambda i: (i, 0)),
          pl.BlockSpec((1, D), lambda i: (0, 0)),
          pl.BlockSpec((D, D), lambda i: (0, 0)),
      ],
      out_specs=pl.BlockSpec((ROWS_TC, D), lambda i: (i, 0)),
      out_shape=jax.ShapeDtypeStruct((NP, D), jnp.float32),
  )(p0, p1, lin, dis_col, b_2d, w)


def _tc_final(p0, p1, lin2, dis_col, b_2d, cw2, cb_2d, fw, fb_2d):
  def body(p0_ref, p1_ref, lin_ref, sn_ref, b_ref, cw_ref, cb_ref, fw_ref,
           fb_ref, out_ref):
    sn = sn_ref[...]
    h = p0_ref[...] + p1_ref[...] + lin_ref[...] * (sn * sn) + b_ref[...]
    h = jnp.maximum(h, 0.0)
    ridx = lax.broadcasted_iota(jnp.int32, (NP, D), 0)
    h = jnp.where(ridx < N, h, 0.0)
    h_prev = jnp.where(ridx == 0, 0.0, pltpu.roll(h, 1, 0))
    h_next = jnp.where(ridx == NP - 1, 0.0, pltpu.roll(h, NP - 1, 0))
    hs = (h_prev, h, h_next)
    t = cb_ref[...].astype(jnp.float32)
    for k in range(3):
      t = t + lax.dot_general(
          hs[k].astype(jnp.bfloat16),
          cw_ref[k * D:(k + 1) * D, :].astype(jnp.bfloat16),
          (((1,), (1,)), ((), ())), preferred_element_type=jnp.float32)
    out_ref[...] = lax.dot_general(
        t.astype(jnp.bfloat16), fw_ref[...].astype(jnp.bfloat16),
        (((1,), (1,)), ((), ())),
        preferred_element_type=jnp.float32) + fb_ref[0, 0]

  return pl.pallas_call(
      body,
      out_shape=jax.ShapeDtypeStruct((NP, 8), jnp.float32),
  )(p0, p1, lin2, dis_col, b_2d, cw2, cb_2d, fw, fb_2d)


def kernel(x, edge_index, edge_weight, W1, b1, W2, b2, conv_w, conv_b,
           fc_w, fc_b):
  e_cnt = edge_weight.shape[0]
  c_ch = 4 * (-(-e_cnt // (NW * B * 4)))
  ept = c_ch * B
  pad = ept * NW - e_cnt

  row = edge_index[0].astype(jnp.int32)
  col = edge_index[1].astype(jnp.int32)
  ew = edge_weight.astype(jnp.float32)
  row3 = jnp.pad(row, (0, pad)).reshape(NW, c_ch, B)
  col3 = jnp.pad(col, (0, pad)).reshape(NW, c_ch, B)
  ew3 = jnp.pad(ew, (0, pad)).reshape(NW, c_ch, B)

  cap = NS * (Q0 + Q1) * B
  cmax = max(Q0, Q1)

  def _split(a):
    ap = jnp.pad(a, (0, cap - e_cnt))
    a0 = ap[:NS * Q0 * B].reshape(NS, Q0, B)
    a1 = ap[NS * Q0 * B:].reshape(NS, Q1, B)
    a0 = jnp.pad(a0, ((0, 0), (0, cmax - Q0), (0, 0)))
    a1 = jnp.pad(a1, ((0, 0), (0, cmax - Q1), (0, 0)))
    return jnp.concatenate([a0, a1], axis=0)

  pk = jnp.stack(
      [_split(row), _split(col),
       _split(lax.bitcast_convert_type(ew, jnp.int32))], axis=2)
  x_p = jnp.pad(x, ((0, NP - N), (0, 0)))
  z2 = jnp.zeros((STRIPE, D), jnp.float32)
  z1 = jnp.zeros((STRIPE,), jnp.float32)

  _DBG = 0

  def _jnp_prop(lin):
    nrm = dis_flat[row] * ew * dis_flat[col]
    p = jnp.zeros((NP, D), jnp.float32).at[col].add(lin[row] * nrm[:, None])
    return jnp.stack([p, jnp.zeros_like(p)])

  degp = _sc_degree(col3, ew3, z1, c_ch)
  if _DBG in (2, 3):
    dg = jnp.zeros((NP,), jnp.float32).at[col].add(ew)
    degp = jnp.stack([dg, jnp.zeros_like(dg)])
  lin1, dis2d = _tc_lin_dis(x_p, W1, degp.reshape(NC, NP // 128, 128))
  dis_flat = dis2d.reshape(NP)
  dis_col = dis2d.reshape(NP, 1)

  if _DBG in (1, 3):
    part1 = _jnp_prop(lin1)
  else:
    part1 = _sc_propagate(lin1, pk, dis_flat, z2, Q0, Q1)
  lin2 = _tc_combine_matmul(part1[0], part1[1], lin1, dis_col,
                            b1.reshape(1, D), W2)
  if _DBG in (1, 3):
    part2 = _jnp_prop(lin2)
  else:
    part2 = _sc_propagate(lin2, pk, dis_flat, z2, Q0, Q1)
  cw2 = jnp.moveaxis(conv_w, 2, 0).reshape(3 * D, D)
  fw8 = jnp.pad(fc_w, ((0, 7), (0, 0)))
  res = _tc_final(part2[0], part2[1], lin2, dis_col, b2.reshape(1, D),
                  cw2, conv_b.reshape(1, D), fw8, fc_b.reshape(1, 1))
  return res[:N, 0]
